# Initial kernel scaffold; baseline (speedup 1.0000x reference)
import functools

import jax
import jax.numpy as jnp
from jax import lax
from jax.experimental import pallas as pl
from jax.experimental.pallas import tpu as pltpu
from jax.experimental.pallas import tpu_sc as plsc

D = 64

_LOG1P_COEF = (
    2.2159764900830936e-07,
    0.9999702432977317,
    -0.49933394898194267,
    0.32751171370201704,
    -0.22396689943036463,
    0.13198966240066795,
    -0.05326747773448861,
    0.01024382863145101,
)


def _softplus16(v):
    t = jnp.exp(-jnp.abs(v))
    p = jnp.float32(_LOG1P_COEF[-1])
    for c in reversed(_LOG1P_COEF[:-1]):
        p = p * t + jnp.float32(c)
    return jnp.maximum(v, jnp.float32(0.0)) + p


@functools.lru_cache(maxsize=None)
def _make(n_idx):
    info = plsc.get_sparse_core_info()
    nc, ns = info.num_cores, info.num_subcores
    nw = nc * ns
    assert n_idx % nw == 0
    per_w = n_idx // nw
    chunk = 512
    assert per_w % chunk == 0
    n_chunks = per_w // chunk
    mesh = plsc.VectorSubcoreMesh(core_axis_name="c", subcore_axis_name="s")

    @functools.partial(
        pl.kernel,
        mesh=mesh,
        out_type=jax.ShapeDtypeStruct((n_idx, D), jnp.float32),
        scratch_types=[
            pltpu.VMEM((per_w,), jnp.int32),
            pltpu.VMEM((chunk, D), jnp.float32),
            pltpu.SemaphoreType.DMA,
        ],
    )
    def k(word_hbm, table_hbm, out_hbm, idx_v, rows_v, sem):
        wid = lax.axis_index("s") * nc + lax.axis_index("c")
        base = wid * per_w
        pltpu.sync_copy(word_hbm.at[pl.ds(base, per_w)], idx_v)

        def do_chunk(c, carry):
            off = c * chunk
            pltpu.async_copy(
                table_hbm.at[idx_v.at[pl.ds(off, chunk)]], rows_v, sem
            ).wait()

            def row(i, carry2):
                for j in range(D // 16):
                    sl = pl.ds(j * 16, 16)
                    rows_v[i, sl] = _softplus16(rows_v[i, sl])
                return carry2

            lax.fori_loop(0, chunk, row, 0)
            pltpu.sync_copy(rows_v, out_hbm.at[pl.ds(base + off, chunk)])
            return carry

        lax.fori_loop(0, n_chunks, do_chunk, 0)

    return k


def kernel(word, emb_weight):
    b, l = word.shape
    flat = word.reshape(-1).astype(jnp.int32)
    out = _make(flat.shape[0])(flat, emb_weight)
    return out.reshape(b, l, D)


# speedup vs baseline: 1.1662x; 1.1662x over previous
import functools

import jax
import jax.numpy as jnp
from jax import lax
from jax.experimental import pallas as pl
from jax.experimental.pallas import tpu as pltpu
from jax.experimental.pallas import tpu_sc as plsc

D = 64

_LOG1P_COEF = (
    2.2159764900830936e-07,
    0.9999702432977317,
    -0.49933394898194267,
    0.32751171370201704,
    -0.22396689943036463,
    0.13198966240066795,
    -0.05326747773448861,
    0.01024382863145101,
)


def _softplus16(v):
    t = jnp.exp(-jnp.abs(v))
    p = jnp.float32(_LOG1P_COEF[-1])
    for c in reversed(_LOG1P_COEF[:-1]):
        p = p * t + jnp.float32(c)
    return jnp.maximum(v, jnp.float32(0.0)) + p


@functools.lru_cache(maxsize=None)
def _make(n_idx):
    info = plsc.get_sparse_core_info()
    nc, ns = info.num_cores, info.num_subcores
    nw = nc * ns
    assert n_idx % nw == 0
    per_w = n_idx // nw
    chunk = 512
    assert per_w % chunk == 0
    n_chunks = per_w // chunk
    mesh = plsc.VectorSubcoreMesh(core_axis_name="c", subcore_axis_name="s")

    @functools.partial(
        pl.kernel,
        mesh=mesh,
        out_type=jax.ShapeDtypeStruct((n_idx, D), jnp.float32),
        scratch_types=[
            pltpu.VMEM((per_w,), jnp.int32),
            pltpu.VMEM((chunk, D), jnp.float32),
            pltpu.SemaphoreType.DMA,
        ],
        compiler_params=pltpu.CompilerParams(use_tc_tiling_on_sc=False),
    )
    def k(word_hbm, table_hbm, out_hbm, idx_v, rows_v, sem):
        wid = lax.axis_index("s") * nc + lax.axis_index("c")
        base = wid * per_w
        pltpu.sync_copy(word_hbm.at[pl.ds(base, per_w)], idx_v)

        def do_chunk(c, carry):
            off = c * chunk
            pltpu.async_copy(
                table_hbm.at[idx_v.at[pl.ds(off, chunk)]], rows_v, sem
            ).wait()

            def row(i, carry2):
                for j in range(D // 16):
                    sl = pl.ds(j * 16, 16)
                    rows_v[i, sl] = _softplus16(rows_v[i, sl])
                return carry2

            lax.fori_loop(0, chunk, row, 0)
            pltpu.sync_copy(rows_v, out_hbm.at[pl.ds(base + off, chunk)])
            return carry

        lax.fori_loop(0, n_chunks, do_chunk, 0)

    return k


def kernel(word, emb_weight):
    b, l = word.shape
    flat = word.reshape(-1).astype(jnp.int32)
    out = _make(flat.shape[0])(flat, emb_weight)
    return out.reshape(b, l, D)


# speedup vs baseline: 1.5979x; 1.3702x over previous
import functools

import jax
import jax.numpy as jnp
from jax import lax
from jax.experimental import pallas as pl
from jax.experimental.pallas import tpu as pltpu
from jax.experimental.pallas import tpu_sc as plsc

D = 64

_LOG1P_COEF = (
    1.472065010887924e-06,
    0.9998476974962351,
    -0.49737321615793884,
    0.3157473167579205,
    -0.19035433673298097,
    0.08269123711134978,
    -0.017414077524237504,
)


def _softplus16(v):
    t = jnp.exp(-jnp.abs(v))
    p = jnp.float32(_LOG1P_COEF[-1])
    for c in reversed(_LOG1P_COEF[:-1]):
        p = p * t + jnp.float32(c)
    return jnp.maximum(v, jnp.float32(0.0)) + p


@functools.lru_cache(maxsize=None)
def _make(n_idx):
    info = plsc.get_sparse_core_info()
    nc, ns = info.num_cores, info.num_subcores
    nw = nc * ns
    assert n_idx % nw == 0
    per_w = n_idx // nw
    chunk = 320
    nbuf = 2
    assert per_w % (chunk * nbuf) == 0
    n_chunks = per_w // chunk
    mesh = plsc.VectorSubcoreMesh(core_axis_name="c", subcore_axis_name="s")

    @functools.partial(
        pl.kernel,
        mesh=mesh,
        out_type=jax.ShapeDtypeStruct((n_idx, D), jnp.float32),
        scratch_types=[
            pltpu.VMEM((per_w,), jnp.int32),
            pltpu.VMEM((nbuf, chunk, D), jnp.float32),
            pltpu.VMEM((nbuf, chunk, D), jnp.float32),
            pltpu.SemaphoreType.DMA,
            pltpu.SemaphoreType.DMA,
            pltpu.SemaphoreType.DMA,
            pltpu.SemaphoreType.DMA,
        ],
        compiler_params=pltpu.CompilerParams(use_tc_tiling_on_sc=False),
    )
    def k(word_hbm, table_hbm, out_hbm, idx_v, gbuf, obuf, gs0, gs1, os0, os1):
        gsems = (gs0, gs1)
        osems = (os0, os1)
        wid = lax.axis_index("s") * nc + lax.axis_index("c")
        base = wid * per_w
        pltpu.sync_copy(word_hbm.at[pl.ds(base, per_w)], idx_v)

        def fire_gather(c, b):
            pltpu.async_copy(
                table_hbm.at[idx_v.at[pl.ds(c * chunk, chunk)]], gbuf.at[b], gsems[b]
            )

        def wait_gather(b):
            pltpu.make_async_copy(
                table_hbm.at[idx_v.at[pl.ds(0, chunk)]], gbuf.at[b], gsems[b]
            ).wait()

        def fire_out(c, b):
            pltpu.async_copy(
                obuf.at[b], out_hbm.at[pl.ds(base + c * chunk, chunk)], osems[b]
            )

        def wait_out(b):
            pltpu.make_async_copy(
                obuf.at[b], out_hbm.at[pl.ds(base, chunk)], osems[b]
            ).wait()

        def compute(b):
            @plsc.parallel_loop(0, chunk, unroll=2)
            def row(i):
                for j in range(D // 16):
                    sl = pl.ds(j * 16, 16)
                    obuf[b, i, sl] = _softplus16(gbuf[b, i, sl])

        for b in range(nbuf):
            fire_gather(b, b)

        for b in range(nbuf):
            wait_gather(b)
            compute(b)
            fire_gather(nbuf + b, b)
            fire_out(b, b)

        @pl.loop(nbuf, n_chunks - nbuf, step=nbuf)
        def grp(c0):
            for b in range(nbuf):
                c = c0 + b
                wait_gather(b)
                wait_out(b)
                compute(b)
                fire_gather(c + nbuf, b)
                fire_out(c, b)

        for b in range(nbuf):
            wait_gather(b)
            wait_out(b)
            compute(b)
            fire_out(n_chunks - nbuf + b, b)

        for b in range(nbuf):
            wait_out(b)

    return k


def kernel(word, emb_weight):
    b, l = word.shape
    flat = word.reshape(-1).astype(jnp.int32)
    out = _make(flat.shape[0])(flat, emb_weight)
    return out.reshape(b, l, D)
